# baseline (speedup 1.0000x reference)
import jax
import jax.numpy as jnp
from jax.experimental import pallas as pl
from jax.experimental.pallas import tpu as pltpu

VOCAB_SIZE = 1000


def _onehot_kernel(idxv_ref, bank_ref, out_ref):
    n_words, k = bank_ref.shape
    ns = idxv_ref.shape[-1]
    vocab = out_ref.shape[-1]
    idxv = idxv_ref[0]
    kio = jax.lax.broadcasted_iota(jnp.int32, (k, 1), 0)
    e = (kio == idxv).astype(jnp.float32)
    toks = jax.lax.dot(bank_ref[...], e,
                       precision=jax.lax.Precision.HIGHEST,
                       preferred_element_type=jnp.float32)
    toks = toks.astype(jnp.int32)
    col = jax.lax.broadcasted_iota(jnp.int32, (n_words, vocab), 1)
    for j in range(ns):
        tok_col = toks[:, j:j + 1]
        out_ref[j * n_words:(j + 1) * n_words, :] = (
            tok_col == col).astype(jnp.float32)


def kernel(features, reports, sentence_bank, weights):
    B, n_sentences, n_words = reports.shape
    k_first, bank_w = sentence_bank.shape

    key = jax.random.key(42)
    idx = jax.random.categorical(key, jnp.log(weights), shape=(B, n_sentences))
    idx = idx.astype(jnp.int32)

    if bank_w < n_words:
        sentence_bank = jnp.pad(sentence_bank, ((0, 0), (0, n_words - bank_w)))
    bank_t = sentence_bank[:, :n_words].T.astype(jnp.float32)

    bb = 2
    ns = bb * n_sentences
    grid = (B // bb,)
    idxv = idx.reshape(B // bb, 1, ns)
    out = pl.pallas_call(
        _onehot_kernel,
        grid=grid,
        in_specs=[
            pl.BlockSpec((1, 1, ns), lambda i: (i, 0, 0)),
            pl.BlockSpec((n_words, k_first), lambda i: (0, 0)),
        ],
        out_specs=pl.BlockSpec((ns * n_words, VOCAB_SIZE), lambda i: (i, 0)),
        out_shape=jax.ShapeDtypeStruct(
            (B * n_sentences * n_words, VOCAB_SIZE), jnp.float32
        ),
    )(idxv, bank_t)

    out = out.reshape(B, n_sentences, n_words, VOCAB_SIZE)
    stops = jnp.zeros((B, n_sentences), dtype=jnp.float32)
    return (out, stops)


# speedup vs baseline: 1.6187x; 1.6187x over previous
import jax
import jax.numpy as jnp
from jax.experimental import pallas as pl
from jax.experimental.pallas import tpu as pltpu

VOCAB_SIZE = 1000
_NSEM = 8
_WINDOW = 128


def _make_kernel(n_total, n_sentences):
    def _kern(idx_ref, bank_ref, out_ref, oh_ref, sems):
        n_words, k = bank_ref.shape
        vocab = oh_ref.shape[-1]
        col = jax.lax.broadcasted_iota(jnp.int32, (n_words, vocab), 1)
        bank_i = bank_ref[...].astype(jnp.int32)
        for kk in range(k):
            tok_col = bank_i[:, kk:kk + 1]
            oh_ref[kk] = (tok_col == col).astype(jnp.float32)

        def copy(i):
            kk = idx_ref[i // n_sentences, i % n_sentences]
            return pltpu.make_async_copy(
                oh_ref.at[kk], out_ref.at[i], sems.at[i % _NSEM])

        for i in range(min(_WINDOW, n_total)):
            copy(i).start()
        for i in range(n_total):
            j = i + _WINDOW
            if j < n_total:
                copy(j).start()
            copy(i).wait()

    return _kern


def kernel(features, reports, sentence_bank, weights):
    B, n_sentences, n_words = reports.shape
    k_first, bank_w = sentence_bank.shape

    key = jax.random.key(42)
    idx = jax.random.categorical(key, jnp.log(weights), shape=(B, n_sentences))
    idx = idx.astype(jnp.int32)

    if bank_w < n_words:
        sentence_bank = jnp.pad(sentence_bank, ((0, 0), (0, n_words - bank_w)))
    bank_t = sentence_bank[:, :n_words].T.astype(jnp.float32)

    n_total = B * n_sentences
    out = pl.pallas_call(
        _make_kernel(n_total, n_sentences),
        in_specs=[
            pl.BlockSpec(memory_space=pltpu.SMEM),
            pl.BlockSpec(memory_space=pltpu.VMEM),
        ],
        out_specs=pl.BlockSpec(memory_space=pl.ANY),
        out_shape=jax.ShapeDtypeStruct(
            (n_total, n_words, VOCAB_SIZE), jnp.float32
        ),
        scratch_shapes=[
            pltpu.VMEM((k_first, n_words, VOCAB_SIZE), jnp.float32),
            pltpu.SemaphoreType.DMA((_NSEM,)),
        ],
    )(idx, bank_t)

    out = out.reshape(B, n_sentences, n_words, VOCAB_SIZE)
    stops = jnp.zeros((B, n_sentences), dtype=jnp.float32)
    return (out, stops)
